# baseline (speedup 1.0000x reference)
import jax
import jax.numpy as jnp
from jax.experimental import pallas as pl
from jax.experimental.pallas import tpu as pltpu

_NH = 32
_S = 8192
_D = 128
_Q = 16
_HC = 4
_NCH = _NH // _HC


def _kv_update_body(pos_ref, ks_ref, vs_ref, kc_ref, vc_ref,
                    ko_ref, vo_ref, bulk_sem, row_sem):
    bulk = []
    for c in range(_NCH):
        h0 = c * _HC
        for i, (src, dst) in enumerate(((kc_ref, ko_ref), (vc_ref, vo_ref))):
            cp = pltpu.make_async_copy(
                src.at[:, pl.ds(h0, _HC)],
                dst.at[:, pl.ds(h0, _HC)],
                bulk_sem.at[i, c],
            )
            cp.start()
            bulk.append(cp)
    for cp in bulk:
        cp.wait()
    rows = []
    for j in range(_Q):
        p = pos_ref[j]
        for i, (src, dst) in enumerate(((ks_ref, ko_ref), (vs_ref, vo_ref))):
            cp = pltpu.make_async_copy(
                src.at[:, :, pl.ds(j, 1), :],
                dst.at[:, :, pl.ds(p, 1), :],
                row_sem.at[i, j],
            )
            cp.start()
            rows.append(cp)
    for cp in rows:
        cp.wait()


def kernel(key_states, value_states, cache_position, key_cache, value_cache):
    return pl.pallas_call(
        _kv_update_body,
        out_shape=(
            jax.ShapeDtypeStruct(key_cache.shape, key_cache.dtype),
            jax.ShapeDtypeStruct(value_cache.shape, value_cache.dtype),
        ),
        in_specs=[
            pl.BlockSpec(memory_space=pltpu.SMEM),
            pl.BlockSpec(memory_space=pl.ANY),
            pl.BlockSpec(memory_space=pl.ANY),
            pl.BlockSpec(memory_space=pl.ANY),
            pl.BlockSpec(memory_space=pl.ANY),
        ],
        out_specs=(
            pl.BlockSpec(memory_space=pl.ANY),
            pl.BlockSpec(memory_space=pl.ANY),
        ),
        scratch_shapes=[
            pltpu.SemaphoreType.DMA((2, _NCH)),
            pltpu.SemaphoreType.DMA((2, _Q)),
        ],
    )(cache_position, key_states, value_states, key_cache, value_cache)


# speedup vs baseline: 42.6135x; 42.6135x over previous
import jax
import jax.numpy as jnp
from jax.experimental import pallas as pl
from jax.experimental.pallas import tpu as pltpu

_NH = 32
_S = 8192
_D = 128
_Q = 16
_SB = 2048
_NSB = _S // _SB


def _kv_update_body(pos_ref, ks_ref, vs_ref, kc_ref, vc_ref, ko_ref, vo_ref):
    s0 = pl.program_id(1) * _SB
    ko_ref[...] = kc_ref[...]
    vo_ref[...] = vc_ref[...]
    for j in range(_Q):
        p = pos_ref[j]
        off = p - s0

        @pl.when(jnp.logical_and(off >= 0, off < _SB))
        def _():
            ko_ref[0, 0, pl.ds(off, 1), :] = ks_ref[0, 0, pl.ds(j, 1), :]
            vo_ref[0, 0, pl.ds(off, 1), :] = vs_ref[0, 0, pl.ds(j, 1), :]


def kernel(key_states, value_states, cache_position, key_cache, value_cache):
    cache_spec = pl.BlockSpec(
        (1, 1, _SB, _D), lambda h, s: (0, h, s, 0))
    states_spec = pl.BlockSpec(
        (1, 1, _Q, _D), lambda h, s: (0, h, 0, 0))
    return pl.pallas_call(
        _kv_update_body,
        grid=(_NH, _NSB),
        out_shape=(
            jax.ShapeDtypeStruct(key_cache.shape, key_cache.dtype),
            jax.ShapeDtypeStruct(value_cache.shape, value_cache.dtype),
        ),
        in_specs=[
            pl.BlockSpec(memory_space=pltpu.SMEM),
            states_spec,
            states_spec,
            cache_spec,
            cache_spec,
        ],
        out_specs=(cache_spec, cache_spec),
        compiler_params=pltpu.CompilerParams(
            dimension_semantics=("arbitrary", "arbitrary"),
        ),
    )(cache_position, key_states, value_states, key_cache, value_cache)
